# Initial kernel scaffold; baseline (speedup 1.0000x reference)
import jax
import jax.numpy as jnp
from jax.experimental import pallas as pl


def kernel(features, edge_index, descriptors, W1, b1, W2, b2, L1w, L1b, L2w, L2b, Cw, Cb):
    raise NotImplementedError("write your pallas kernel here")


# speedup vs baseline: 7.6280x; 7.6280x over previous
import functools

import jax
import jax.numpy as jnp
from jax import lax
from jax.experimental import pallas as pl
from jax.experimental.pallas import tpu as pltpu
from jax.experimental.pallas import tpu_sc as plsc

N_NODES = 10000
N_EDGES = 320000
D = 128

NC = 2
NS = 16
NW = NC * NS
EDGES_PER_TILE = N_EDGES // NW
BATCH = 125
NB = EDGES_PER_TILE // BATCH
N_PAD = 10240
ROWS_PER_TILE = N_PAD // NS


def _sc_agg_body(y, srcm, dstm, zeros, out, srcv, dstv, rows, acc, sem):
    c = lax.axis_index("c")
    s = lax.axis_index("s")
    wid = c * NS + s
    base = wid * NB
    pltpu.sync_copy(srcm.at[pl.ds(base, NB)], srcv)
    pltpu.sync_copy(dstm.at[pl.ds(base, NB)], dstv)
    pltpu.sync_copy(zeros, acc.at[pl.ds(s * ROWS_PER_TILE, ROWS_PER_TILE)])
    plsc.subcore_barrier()

    def body(b, carry):
        pltpu.async_copy(y.at[srcv.at[b]], rows, sem).wait()
        pltpu.sync_copy(rows, acc.at[dstv.at[b]], add=True)
        return carry

    lax.fori_loop(0, NB, body, 0)
    plsc.subcore_barrier()
    pltpu.sync_copy(acc.at[pl.ds(s * ROWS_PER_TILE, ROWS_PER_TILE)],
                    out.at[c, pl.ds(s * ROWS_PER_TILE, ROWS_PER_TILE)])


_sc_agg = pl.kernel(
    _sc_agg_body,
    out_type=jax.ShapeDtypeStruct((NC, N_PAD, D), jnp.float32),
    mesh=plsc.VectorSubcoreMesh(core_axis_name="c", subcore_axis_name="s"),
    scratch_types=[
        pltpu.VMEM((NB, BATCH), jnp.int32),
        pltpu.VMEM((NB, BATCH), jnp.int32),
        pltpu.VMEM((BATCH, D), jnp.float32),
        pltpu.VMEM_SHARED((N_PAD, D), jnp.float32),
        pltpu.SemaphoreType.DMA,
    ],
)


def _mm_body(x_ref, w_ref, o_ref):
    o_ref[...] = jnp.dot(x_ref[...], w_ref[...],
                         preferred_element_type=jnp.float32)


_mm1 = pl.pallas_call(
    _mm_body,
    grid=(5,),
    in_specs=[pl.BlockSpec((2000, D), lambda i: (i, 0)),
              pl.BlockSpec((D, D), lambda i: (0, 0))],
    out_specs=pl.BlockSpec((2000, D), lambda i: (i, 0)),
    out_shape=jax.ShapeDtypeStruct((N_NODES, D), jnp.float32),
)


def _fuse_body(p_ref, b_ref, w_ref, o_ref):
    h = jnp.maximum(p_ref[0] + p_ref[1] + b_ref[...], 0.0)
    o_ref[...] = jnp.dot(h, w_ref[...], preferred_element_type=jnp.float32)


_fuse2 = pl.pallas_call(
    _fuse_body,
    grid=(5,),
    in_specs=[pl.BlockSpec((NC, 2000, D), lambda i: (0, i, 0)),
              pl.BlockSpec((1, D), lambda i: (0, 0)),
              pl.BlockSpec((D, D), lambda i: (0, 0))],
    out_specs=pl.BlockSpec((2000, D), lambda i: (i, 0)),
    out_shape=jax.ShapeDtypeStruct((N_NODES, D), jnp.float32),
)


def _head_body(p_ref, b2_ref, desc_ref, l1wa_ref, l1wb_ref, l1b_ref,
               l2w_ref, l2b_ref, cw_ref, cb_ref, o_ref, acc_ref):
    g = pl.program_id(0)

    @pl.when(g == 0)
    def _init():
        acc_ref[...] = jnp.zeros_like(acc_ref)

    h = jnp.maximum(p_ref[0] + p_ref[1] + b2_ref[...], 0.0)
    acc_ref[...] += jnp.sum(h, axis=0, keepdims=True)

    @pl.when(g == pl.num_programs(0) - 1)
    def _finish():
        hg = acc_ref[...] * (1.0 / N_NODES)
        t = (jnp.dot(hg, l1wa_ref[...], preferred_element_type=jnp.float32)
             + jnp.dot(desc_ref[...], l1wb_ref[...],
                       preferred_element_type=jnp.float32)
             + l1b_ref[...])
        t = jnp.maximum(t, 0.0)
        t = jnp.maximum(
            jnp.dot(t, l2w_ref[...], preferred_element_type=jnp.float32)
            + l2b_ref[...], 0.0)
        o_ref[...] = (jnp.dot(t, cw_ref[...],
                              preferred_element_type=jnp.float32)
                      + cb_ref[...])


_head = pl.pallas_call(
    _head_body,
    grid=(10,),
    in_specs=[pl.BlockSpec((NC, 1000, D), lambda i: (0, i, 0)),
              pl.BlockSpec((1, D), lambda i: (0, 0)),
              pl.BlockSpec((1, 16), lambda i: (0, 0)),
              pl.BlockSpec((D, 500), lambda i: (0, 0)),
              pl.BlockSpec((16, 500), lambda i: (0, 0)),
              pl.BlockSpec((1, 500), lambda i: (0, 0)),
              pl.BlockSpec((500, 100), lambda i: (0, 0)),
              pl.BlockSpec((1, 100), lambda i: (0, 0)),
              pl.BlockSpec((100, 10), lambda i: (0, 0)),
              pl.BlockSpec((1, 10), lambda i: (0, 0))],
    out_specs=pl.BlockSpec((1, 10), lambda i: (0, 0)),
    out_shape=jax.ShapeDtypeStruct((1, 10), jnp.float32),
    scratch_shapes=[pltpu.VMEM((1, D), jnp.float32)],
)


def kernel(features, edge_index, descriptors,
           W1, b1, W2, b2, L1w, L1b, L2w, L2b, Cw, Cb):
    ei = edge_index.astype(jnp.int32)
    srcm = ei[0].reshape(NW * NB, BATCH)
    dstm = ei[1].reshape(NW * NB, BATCH)
    zeros = jnp.zeros((ROWS_PER_TILE, D), jnp.float32)

    y1 = _mm1(features, W1)
    p1 = _sc_agg(y1, srcm, dstm, zeros)
    y2 = _fuse2(p1, b1.reshape(1, D), W2)
    p2 = _sc_agg(y2, srcm, dstm, zeros)
    return _head(p2, b2.reshape(1, D), descriptors,
                 L1w[:D], L1w[D:], L1b.reshape(1, 500),
                 L2w, L2b.reshape(1, 100), Cw, Cb.reshape(1, 10))
